# baseline (speedup 1.0000x reference)
import functools

import jax
import jax.numpy as jnp
from jax import lax
from jax.experimental import pallas as pl
from jax.experimental.pallas import tpu as pltpu
from jax.experimental.pallas import tpu_sc as plsc

B = 4096
L = 200
D = 128
N = B * L
NC = 2
NS = 16
NW = NC * NS
PER_W = N // NW
CHUNK = 256
NCHUNK = PER_W // CHUNK
NPAIR = NCHUNK // 2

_mesh = plsc.VectorSubcoreMesh(core_axis_name="c", subcore_axis_name="s")


@functools.partial(
    pl.kernel,
    mesh=_mesh,
    out_type=jax.ShapeDtypeStruct((N, D), jnp.float32),
    scratch_types=[
        pltpu.VMEM((PER_W,), jnp.int32),
        pltpu.VMEM((CHUNK, D), jnp.float32),
        pltpu.VMEM((CHUNK, D), jnp.float32),
        pltpu.VMEM_SHARED((NS * CHUNK, D), jnp.float32),
        pltpu.SemaphoreType.DMA,
        pltpu.SemaphoreType.DMA,
        pltpu.SemaphoreType.DMA,
        pltpu.SemaphoreType.DMA,
    ],
)
def _gather_kernel(idx_hbm, table_hbm, out_hbm, idx_v,
                   b0, b1, sp0, gs0, gs1, dws, ws):
    sid = lax.axis_index("s")
    wid = sid * NC + lax.axis_index("c")
    base = wid * PER_W
    s0 = sp0.at[pl.ds(sid * CHUNK, CHUNK)]
    pltpu.sync_copy(idx_hbm.at[pl.ds(base, PER_W)], idx_v)
    pltpu.async_copy(table_hbm.at[idx_v.at[pl.ds(0, CHUNK)]], b0, gs0)

    def body(j, carry):
        e = j * 2
        o = e + 1
        pltpu.async_copy(
            table_hbm.at[idx_v.at[pl.ds(o * CHUNK, CHUNK)]], b1, gs1)
        pltpu.make_async_copy(
            table_hbm.at[idx_v.at[pl.ds(e * CHUNK, CHUNK)]], b0, gs0).wait()
        pltpu.async_copy(b0, out_hbm.at[pl.ds(base + e * CHUNK, CHUNK)], dws)

        @pl.when(j > 0)
        def _():
            pltpu.make_async_copy(
                b0, out_hbm.at[pl.ds(base + (e - 2) * CHUNK, CHUNK)],
                dws).wait()

        @pl.when(j + 1 < NPAIR)
        def _():
            pltpu.async_copy(
                table_hbm.at[idx_v.at[pl.ds((e + 2) * CHUNK, CHUNK)]], b0, gs0)

        pltpu.make_async_copy(
            table_hbm.at[idx_v.at[pl.ds(o * CHUNK, CHUNK)]], b1, gs1).wait()

        @pl.when(j > 0)
        def _():
            pltpu.make_async_copy(
                s0, out_hbm.at[pl.ds(base + (o - 2) * CHUNK, CHUNK)],
                ws).wait()

        pltpu.sync_copy(b1, s0)
        pltpu.async_copy(s0, out_hbm.at[pl.ds(base + o * CHUNK, CHUNK)], ws)

        @pl.when(j + 1 < NPAIR)
        def _():
            pltpu.async_copy(
                table_hbm.at[idx_v.at[pl.ds((o + 2) * CHUNK, CHUNK)]], b1, gs1)
        return carry

    lax.fori_loop(0, NPAIR, body, 0)
    e_last = (NPAIR - 1) * 2
    pltpu.make_async_copy(
        b0, out_hbm.at[pl.ds(base + e_last * CHUNK, CHUNK)], dws).wait()
    pltpu.make_async_copy(
        s0, out_hbm.at[pl.ds(base + (e_last + 1) * CHUNK, CHUNK)], ws).wait()


def kernel(x, table):
    out = _gather_kernel(x.reshape(-1), table)
    return out.reshape(B, L, D)


# speedup vs baseline: 1.2442x; 1.2442x over previous
import functools

import jax
import jax.numpy as jnp
from jax import lax
from jax.experimental import pallas as pl
from jax.experimental.pallas import tpu as pltpu
from jax.experimental.pallas import tpu_sc as plsc

B = 4096
L = 200
D = 128
N = B * L
NC = 2
NS = 16
NW = NC * NS
PER_W = N // NW
CHUNK = 200
NCHUNK = PER_W // CHUNK
NPAIR = NCHUNK // 2

_mesh = plsc.VectorSubcoreMesh(core_axis_name="c", subcore_axis_name="s")


@functools.partial(
    pl.kernel,
    mesh=_mesh,
    out_type=jax.ShapeDtypeStruct((N, D), jnp.float32),
    scratch_types=[
        pltpu.VMEM((PER_W,), jnp.int32),
        pltpu.VMEM((CHUNK, D), jnp.float32),
        pltpu.VMEM((CHUNK, D), jnp.float32),
        pltpu.VMEM_SHARED((NS * CHUNK, D), jnp.float32),
        pltpu.VMEM_SHARED((NS * CHUNK, D), jnp.float32),
        pltpu.SemaphoreType.DMA,
        pltpu.SemaphoreType.DMA,
        pltpu.SemaphoreType.DMA,
        pltpu.SemaphoreType.DMA,
    ],
)
def _gather_kernel(idx_hbm, table_hbm, out_hbm, idx_v,
                   b0, b1, sp0, sp1, gs0, gs1, w0, w1):
    sid = lax.axis_index("s")
    wid = sid * NC + lax.axis_index("c")
    base = wid * PER_W
    srow = sid * CHUNK
    s0 = sp0.at[pl.ds(srow, CHUNK)]
    s1 = sp1.at[pl.ds(srow, CHUNK)]
    pltpu.sync_copy(idx_hbm.at[pl.ds(base, PER_W)], idx_v)
    pltpu.async_copy(table_hbm.at[idx_v.at[pl.ds(0, CHUNK)]], b0, gs0)

    def body(j, carry):
        g0 = j * 2
        pltpu.async_copy(
            table_hbm.at[idx_v.at[pl.ds((g0 + 1) * CHUNK, CHUNK)]], b1, gs1)
        pltpu.make_async_copy(
            table_hbm.at[idx_v.at[pl.ds(g0 * CHUNK, CHUNK)]], b0, gs0).wait()

        @pl.when(j > 0)
        def _():
            pltpu.make_async_copy(
                s0, out_hbm.at[pl.ds(base + (g0 - 2) * CHUNK, CHUNK)], w0).wait()

        pltpu.sync_copy(b0, s0)
        pltpu.async_copy(s0, out_hbm.at[pl.ds(base + g0 * CHUNK, CHUNK)], w0)

        @pl.when(j + 1 < NPAIR)
        def _():
            pltpu.async_copy(
                table_hbm.at[idx_v.at[pl.ds((g0 + 2) * CHUNK, CHUNK)]], b0, gs0)

        pltpu.make_async_copy(
            table_hbm.at[idx_v.at[pl.ds((g0 + 1) * CHUNK, CHUNK)]], b1, gs1).wait()

        @pl.when(j > 0)
        def _():
            pltpu.make_async_copy(
                s1, out_hbm.at[pl.ds(base + (g0 - 1) * CHUNK, CHUNK)], w1).wait()

        pltpu.sync_copy(b1, s1)
        pltpu.async_copy(s1, out_hbm.at[pl.ds(base + (g0 + 1) * CHUNK, CHUNK)], w1)
        return carry

    lax.fori_loop(0, NPAIR, body, 0)
    g_last = (NPAIR - 1) * 2
    pltpu.make_async_copy(
        s0, out_hbm.at[pl.ds(base + g_last * CHUNK, CHUNK)], w0).wait()
    pltpu.make_async_copy(
        s1, out_hbm.at[pl.ds(base + (g_last + 1) * CHUNK, CHUNK)], w1).wait()


def kernel(x, table):
    out = _gather_kernel(x.reshape(-1), table)
    return out.reshape(B, L, D)
